# baseline (speedup 1.0000x reference)
import functools

import jax
import jax.numpy as jnp
from jax import lax
from jax.experimental import pallas as pl
from jax.experimental.pallas import tpu as pltpu
from jax.experimental.pallas import tpu_sc as plsc

N = 10000
E = 320000
D = 128
NC = 2
NS = 16
NW = NC * NS
EPW = E // NW
K = 80
CHUNKS = EPW // K
NB = 3
ROUNDS = 5
RC = CHUNKS // ROUNDS
NPAD = 10240
RPT = NPAD // NS
FIRE = 25

_mesh = plsc.VectorSubcoreMesh(core_axis_name="c", subcore_axis_name="s")


def _lane_bcast(wgrp, kk):
    return lax.gather(
        wgrp, jnp.full((16,), 1, jnp.int32).reshape(16, 1) * kk,
        lax.GatherDimensionNumbers(
            offset_dims=(), collapsed_slice_dims=(0,), start_index_map=(0,)),
        slice_sizes=(1,),
        mode=lax.GatherScatterMode.PROMISE_IN_BOUNDS)


@functools.partial(
    pl.kernel,
    mesh=_mesh,
    out_type=jax.ShapeDtypeStruct((NC, NPAD), jnp.float32),
    scratch_types=[
        pltpu.VMEM((CHUNKS, K), jnp.int32),
        pltpu.VMEM((CHUNKS, K), jnp.float32),
        pltpu.VMEM_SHARED((NPAD,), jnp.float32),
        pltpu.SemaphoreType.DMA,
    ],
)
def _deg_kernel(col_hbm, ew_hbm, z1_hbm, deg_hbm, col_all, ew_all, acc_sh,
                dsem):
    c = lax.axis_index("c")
    s = lax.axis_index("s")
    wid = c * NS + s
    pltpu.sync_copy(z1_hbm, acc_sh.at[pl.ds(s * RPT, RPT)])
    pltpu.sync_copy(col_hbm.at[wid], col_all)
    pltpu.sync_copy(ew_hbm.at[wid], ew_all)
    plsc.subcore_barrier()

    def rnd(r, carry):
        for q in range(FIRE):
            ci = r * FIRE + q
            pltpu.async_copy(ew_all.at[ci], acc_sh.at[col_all.at[ci]], dsem,
                             add=True)
        for q in range(FIRE):
            ci = r * FIRE + q
            pltpu.make_async_copy(ew_all.at[ci], acc_sh.at[col_all.at[ci]],
                                  dsem).wait()
        return carry

    lax.fori_loop(0, CHUNKS // FIRE, rnd, 0)
    plsc.subcore_barrier()
    pltpu.sync_copy(acc_sh.at[pl.ds(s * RPT, RPT)],
                    deg_hbm.at[c, pl.ds(s * RPT, RPT)])


@functools.partial(
    pl.kernel,
    mesh=_mesh,
    out_type=jax.ShapeDtypeStruct((NC, NPAD, D), jnp.float32),
    scratch_types=[
        pltpu.VMEM_SHARED((NPAD, D), jnp.float32),
        pltpu.VMEM((RC, K), jnp.int32),
        pltpu.VMEM((RC, K), jnp.int32),
        pltpu.VMEM((RC, K), jnp.float32),
    ] + [pltpu.VMEM((K, D), jnp.float32)] * NB
      + [pltpu.SemaphoreType.DMA] * (2 * NB),
)
def _agg_kernel(row_hbm, col_hbm, ew_hbm, y_hbm, z2_hbm, out_hbm,
                acc_sh, row_r, col_r, ew_r,
                buf0, buf1, buf2,
                gs0, gs1, gs2, ss0, ss1, ss2):
    bufs = (buf0, buf1, buf2)
    gsems = (gs0, gs1, gs2)
    ssems = (ss0, ss1, ss2)
    c = lax.axis_index("c")
    s = lax.axis_index("s")
    wid = c * NS + s
    pltpu.sync_copy(z2_hbm, acc_sh.at[pl.ds(s * RPT, RPT)])
    plsc.subcore_barrier()

    def scale(b, i):
        buf = bufs[b]

        def grp(g, carry2):
            wgrp = ew_r[i, pl.ds(g * 16, 16)]

            def four(q, carry3):
                for u in range(4):
                    kk = q * 4 + u
                    wv = _lane_bcast(wgrp, kk)
                    k = g * 16 + kk
                    for j in range(D // 16):
                        sl = pl.ds(j * 16, 16)
                        buf[k, sl] = buf[k, sl] * wv
                return carry3

            lax.fori_loop(0, 4, four, 0)
            return carry2

        lax.fori_loop(0, K // 16, grp, 0)

    def rnd(r, carry):
        pltpu.sync_copy(row_hbm.at[wid, r], row_r)
        pltpu.sync_copy(col_hbm.at[wid, r], col_r)
        pltpu.sync_copy(ew_hbm.at[wid, r], ew_r)
        gh = {0: pltpu.async_copy(y_hbm.at[row_r.at[0]], bufs[0], gsems[0])}
        sh = {}
        for i in range(RC):
            b = i % NB
            gh.pop(i).wait()
            scale(b, i)
            sh[i] = pltpu.async_copy(bufs[b], acc_sh.at[col_r.at[i]],
                                     ssems[b], add=True)
            if i >= 1:
                sh.pop(i - 1).wait()
            if i + 1 < RC:
                nb = (i + 1) % NB
                gh[i + 1] = pltpu.async_copy(y_hbm.at[row_r.at[i + 1]],
                                             bufs[nb], gsems[nb])
        sh.pop(RC - 1).wait()
        return carry

    lax.fori_loop(0, ROUNDS, rnd, 0)
    plsc.subcore_barrier()
    pltpu.sync_copy(acc_sh.at[pl.ds(s * RPT, RPT)],
                    out_hbm.at[c, pl.ds(s * RPT, RPT)])


def _y_body(x_ref, w_ref, degp_ref, y_ref):
    deg = degp_ref[0, :] + degp_ref[1, :] + 1.0
    dis = lax.rsqrt(deg)
    xw = jnp.dot(x_ref[...], w_ref[...], preferred_element_type=jnp.float32)
    y_ref[...] = xw * dis[:N, None]


def _fin_body(p_ref, y_ref, degp_ref, b_ref, o_ref):
    deg = degp_ref[0, :] + degp_ref[1, :] + 1.0
    dis = lax.rsqrt(deg)
    tot = p_ref[0, :N, :] + p_ref[1, :N, :] + y_ref[...]
    o_ref[...] = jnp.maximum(tot * dis[:N, None] + b_ref[...], 0.0)


def kernel(x, edge_idx, edge_weight, W, b):
    ei = edge_idx.astype(jnp.int32)
    row = ei[0]
    col = ei[1]
    ew = edge_weight.astype(jnp.float32)
    z1 = jnp.zeros((NPAD // NS,), jnp.float32)
    z2 = jnp.zeros((RPT, D), jnp.float32)

    degp = _deg_kernel(col.reshape(NW, CHUNKS, K),
                       ew.reshape(NW, CHUNKS, K), z1)
    y = pl.pallas_call(
        _y_body,
        out_shape=jax.ShapeDtypeStruct((N, D), jnp.float32),
    )(x, W, degp)
    p = _agg_kernel(row.reshape(NW, ROUNDS, RC, K),
                    col.reshape(NW, ROUNDS, RC, K),
                    ew.reshape(NW, ROUNDS, RC, K), y, z2)
    out = pl.pallas_call(
        _fin_body,
        out_shape=jax.ShapeDtypeStruct((N, D), jnp.float32),
    )(p, y, degp, b)
    return out


# speedup vs baseline: 1.3874x; 1.3874x over previous
import functools

import jax
import jax.numpy as jnp
from jax import lax
from jax.experimental import pallas as pl
from jax.experimental.pallas import tpu as pltpu
from jax.experimental.pallas import tpu_sc as plsc

N = 10000
E = 320000
D = 128
NC = 2
NS = 16
NW = NC * NS
EPW = E // NW
K = 80
CHUNKS = EPW // K
NB = 3
ROUNDS = 5
RC = CHUNKS // ROUNDS
NPAD = 10240
RPT = NPAD // NS
FIRE = 25

_mesh = plsc.VectorSubcoreMesh(core_axis_name="c", subcore_axis_name="s")


def _lane_bcast(wgrp, kk):
    return lax.gather(
        wgrp, jnp.full((16,), 1, jnp.int32).reshape(16, 1) * kk,
        lax.GatherDimensionNumbers(
            offset_dims=(), collapsed_slice_dims=(0,), start_index_map=(0,)),
        slice_sizes=(1,),
        mode=lax.GatherScatterMode.PROMISE_IN_BOUNDS)


@functools.partial(
    pl.kernel,
    mesh=_mesh,
    out_type=jax.ShapeDtypeStruct((NC, NPAD), jnp.float32),
    scratch_types=[
        pltpu.VMEM((CHUNKS, K), jnp.int32),
        pltpu.VMEM((CHUNKS, K), jnp.float32),
        pltpu.VMEM_SHARED((NPAD,), jnp.float32),
        pltpu.SemaphoreType.DMA,
    ],
)
def _deg_kernel(col_hbm, ew_hbm, z1_hbm, deg_hbm, col_all, ew_all, acc_sh,
                dsem):
    c = lax.axis_index("c")
    s = lax.axis_index("s")
    wid = c * NS + s
    pltpu.sync_copy(z1_hbm, acc_sh.at[pl.ds(s * RPT, RPT)])
    pltpu.sync_copy(col_hbm.at[wid], col_all)
    pltpu.sync_copy(ew_hbm.at[wid], ew_all)
    plsc.subcore_barrier()

    def rnd(r, carry):
        for q in range(FIRE):
            ci = r * FIRE + q
            pltpu.async_copy(ew_all.at[ci], acc_sh.at[col_all.at[ci]], dsem,
                             add=True)
        for q in range(FIRE):
            ci = r * FIRE + q
            pltpu.make_async_copy(ew_all.at[ci], acc_sh.at[col_all.at[ci]],
                                  dsem).wait()
        return carry

    lax.fori_loop(0, CHUNKS // FIRE, rnd, 0)
    plsc.subcore_barrier()
    pltpu.sync_copy(acc_sh.at[pl.ds(s * RPT, RPT)],
                    deg_hbm.at[c, pl.ds(s * RPT, RPT)])


@functools.partial(
    pl.kernel,
    mesh=_mesh,
    out_type=jax.ShapeDtypeStruct((NC, NPAD, D), jnp.float32),
    scratch_types=[
        pltpu.VMEM_SHARED((NPAD, D), jnp.float32),
        pltpu.VMEM((RC, K), jnp.int32),
        pltpu.VMEM((RC, K), jnp.int32),
        pltpu.VMEM((RC, K), jnp.float32),
    ] + [pltpu.VMEM((K, D), jnp.float32)] * NB
      + [pltpu.SemaphoreType.DMA] * (2 * NB),
)
def _agg_kernel(row_hbm, col_hbm, ew_hbm, y_hbm, z2_hbm, out_hbm,
                acc_sh, row_r, col_r, ew_r,
                buf0, buf1, buf2,
                gs0, gs1, gs2, ss0, ss1, ss2):
    bufs = (buf0, buf1, buf2)
    gsems = (gs0, gs1, gs2)
    ssems = (ss0, ss1, ss2)
    c = lax.axis_index("c")
    s = lax.axis_index("s")
    wid = c * NS + s
    pltpu.sync_copy(z2_hbm, acc_sh.at[pl.ds(s * RPT, RPT)])
    plsc.subcore_barrier()

    def scale(b, i):
        buf = bufs[b]

        def grp(g, carry2):
            wgrp = ew_r[i, pl.ds(g * 16, 16)]

            def four(q, carry3):
                for u in range(4):
                    kk = q * 4 + u
                    wv = _lane_bcast(wgrp, kk)
                    k = g * 16 + kk
                    for j in range(D // 16):
                        sl = pl.ds(j * 16, 16)
                        buf[k, sl] = buf[k, sl] * wv
                return carry3

            lax.fori_loop(0, 4, four, 0)
            return carry2

        lax.fori_loop(0, K // 16, grp, 0)

    def rnd(r, carry):
        pltpu.sync_copy(row_hbm.at[wid, r], row_r)
        pltpu.sync_copy(col_hbm.at[wid, r], col_r)
        pltpu.sync_copy(ew_hbm.at[wid, r], ew_r)
        gh = {}
        sh = {}
        for i in range(2):
            gh[i] = pltpu.async_copy(y_hbm.at[row_r.at[i]], bufs[i],
                                     gsems[i])
        for i in range(RC):
            b = i % NB
            gh.pop(i).wait()
            scale(b, i)
            sh[i] = pltpu.async_copy(bufs[b], acc_sh.at[col_r.at[i]],
                                     ssems[b], add=True)
            if i >= 1:
                sh.pop(i - 1).wait()
            if i + 2 < RC:
                nb = (i + 2) % NB
                gh[i + 2] = pltpu.async_copy(y_hbm.at[row_r.at[i + 2]],
                                             bufs[nb], gsems[nb])
        sh.pop(RC - 1).wait()
        return carry

    lax.fori_loop(0, ROUNDS, rnd, 0)
    plsc.subcore_barrier()
    pltpu.sync_copy(acc_sh.at[pl.ds(s * RPT, RPT)],
                    out_hbm.at[c, pl.ds(s * RPT, RPT)])


def _y_body(x_ref, w_ref, degp_ref, y_ref):
    deg = degp_ref[0, :] + degp_ref[1, :] + 1.0
    dis = lax.rsqrt(deg)
    xw = jnp.dot(x_ref[...], w_ref[...], preferred_element_type=jnp.float32)
    y_ref[...] = xw * dis[:N, None]


def _fin_body(p_ref, y_ref, degp_ref, b_ref, o_ref):
    deg = degp_ref[0, :] + degp_ref[1, :] + 1.0
    dis = lax.rsqrt(deg)
    tot = p_ref[0, :N, :] + p_ref[1, :N, :] + y_ref[...]
    o_ref[...] = jnp.maximum(tot * dis[:N, None] + b_ref[...], 0.0)


def kernel(x, edge_idx, edge_weight, W, b):
    ei = edge_idx.astype(jnp.int32)
    row = ei[0]
    col = ei[1]
    ew = edge_weight.astype(jnp.float32)
    z1 = jnp.zeros((NPAD // NS,), jnp.float32)
    z2 = jnp.zeros((RPT, D), jnp.float32)

    degp = _deg_kernel(col.reshape(NW, CHUNKS, K),
                       ew.reshape(NW, CHUNKS, K), z1)
    y = pl.pallas_call(
        _y_body,
        out_shape=jax.ShapeDtypeStruct((N, D), jnp.float32),
    )(x, W, degp)
    p = _agg_kernel(row.reshape(NW, ROUNDS, RC, K),
                    col.reshape(NW, ROUNDS, RC, K),
                    ew.reshape(NW, ROUNDS, RC, K), y, z2)
    out = pl.pallas_call(
        _fin_body,
        out_shape=jax.ShapeDtypeStruct((N, D), jnp.float32),
    )(p, y, degp, b)
    return out
